# baseline (speedup 1.0000x reference)
import jax
import jax.numpy as jnp
from jax.experimental import pallas as pl
from jax.experimental.pallas import tpu as pltpu

_K = 4
_TOPK = 512
_EPS = 0.01
_ROWS = 128


def _kth_bits(zbits):
    rows = zbits.shape[0]
    h = (zbits >> 16).astype(jnp.int16)

    lo = jnp.zeros((rows, 1), jnp.int32)
    hi = jnp.full((rows, 1), 32640, jnp.int32)

    def s1(_, carry):
        lo, hi = carry
        mid = lo + ((hi - lo) >> 1)
        cnt = jnp.sum((h >= mid.astype(jnp.int16)).astype(jnp.int32), axis=1,
                      keepdims=True)
        p = cnt >= _TOPK
        return jnp.where(p, mid, lo), jnp.where(p, hi, mid)

    m, _ = jax.lax.fori_loop(0, 15, s1, (lo, hi))

    m16 = m.astype(jnp.int16)
    c_gt = jnp.sum((h > m16).astype(jnp.int32), axis=1, keepdims=True)
    w = jnp.where(h == m16, (zbits & 0xFFFF) ^ 0x8000, 0x8000).astype(jnp.int16)

    lo2 = jnp.full((rows, 1), -32768, jnp.int32)
    hi2 = jnp.full((rows, 1), 32768, jnp.int32)

    def s2(_, carry):
        lo2, hi2 = carry
        mid = lo2 + ((hi2 - lo2) >> 1)
        cnt = jnp.sum((w >= mid.astype(jnp.int16)).astype(jnp.int32), axis=1,
                      keepdims=True)
        p = c_gt + cnt >= _TOPK
        return jnp.where(p, mid, lo2), jnp.where(p, hi2, mid)

    lo2, _ = jax.lax.fori_loop(0, 16, s2, (lo2, hi2))
    return (m << 16) + (lo2 + 32768)


def _soft_threshold(z, theta):
    zbits = jax.lax.bitcast_convert_type(z, jnp.int32) & 0x7FFFFFFF
    kth = _kth_bits(zbits)
    soft = z - jnp.clip(z, -theta, theta)
    return jnp.where(zbits > kth, z, soft)


def _body(y_ref, A_ref, gamma_ref, theta_ref, a_par_ref, v_ref, vu_ref,
          out_ref):
    y = y_ref[...]
    A = A_ref[...]

    yA = jax.lax.dot_general(y, A, (((1,), (0,)), ((), ())),
                             preferred_element_type=jnp.float32)
    z = gamma_ref[0] * yA
    x_ = _soft_threshold(z, theta_ref[0])
    x = x_ + a_par_ref[0] * (x_ / (jnp.abs(x_) + _EPS))

    for i in range(1, _K):
        tvu = theta_ref[i] * vu_ref[i]
        g = x + tvu * x * jnp.exp(-v_ref[i] * jnp.abs(x))
        a = jax.lax.dot_general(g, A, (((1,), (1,)), ((), ())),
                                preferred_element_type=jnp.float32)
        b = a - y
        c = jax.lax.dot_general(b, A, (((1,), (0,)), ((), ())),
                                preferred_element_type=jnp.float32)
        z = x - gamma_ref[i] * c
        x_ = _soft_threshold(z, theta_ref[i])
        dx = x_ - x
        x = x_ + a_par_ref[i] * (dx / (jnp.abs(dx) + _EPS))

    out_ref[...] = x


@jax.jit
def kernel(y, info, A, gamma, theta, a_par, v, vu, theta_init):
    batch, m = y.shape
    n = A.shape[1]
    smem = pl.BlockSpec(memory_space=pltpu.SMEM)
    x = pl.pallas_call(
        _body,
        grid=(batch // _ROWS,),
        in_specs=[
            pl.BlockSpec((_ROWS, m), lambda i: (i, 0)),
            pl.BlockSpec((m, n), lambda i: (0, 0)),
            smem, smem, smem, smem, smem,
        ],
        out_specs=pl.BlockSpec((_ROWS, n), lambda i: (i, 0)),
        out_shape=jax.ShapeDtypeStruct((batch, n), jnp.float32),
        compiler_params=pltpu.CompilerParams(
            dimension_semantics=("parallel",),
            vmem_limit_bytes=100 * 1024 * 1024,
        ),
    )(y, A, gamma, theta, a_par, v, vu)
    zk = jnp.zeros((_K, 1), jnp.float32)
    return x, zk, zk


# speedup vs baseline: 1.9638x; 1.9638x over previous
import jax
import jax.numpy as jnp
from jax.experimental import pallas as pl
from jax.experimental.pallas import tpu as pltpu

_K = 4
_TOPK = 512
_EPS = 0.01
_ROWS = 128


def _kth_bits(zbits):
    rows = zbits.shape[0]
    lo = jnp.zeros((rows, 1), jnp.int32)
    hi = jnp.full((rows, 1), 0x7F800001, jnp.int32)

    def step(_, carry):
        lo, hi = carry
        mid = lo + ((hi - lo) >> 1)
        cnt = jnp.sum((zbits >= mid).astype(jnp.int32), axis=1, keepdims=True)
        p = cnt >= _TOPK
        return jnp.where(p, mid, lo), jnp.where(p, hi, mid)

    lo, _ = jax.lax.fori_loop(0, 32, step, (lo, hi), unroll=8)
    return lo


def _soft_threshold(z, theta):
    zbits = jax.lax.bitcast_convert_type(z, jnp.int32) & 0x7FFFFFFF
    kth = _kth_bits(zbits)
    soft = z - jnp.clip(z, -theta, theta)
    return jnp.where(zbits > kth, z, soft)


def _body(y_ref, A_ref, gamma_ref, theta_ref, a_par_ref, v_ref, vu_ref,
          out_ref):
    y = y_ref[...]
    A = A_ref[...]

    yA = jax.lax.dot_general(y, A, (((1,), (0,)), ((), ())),
                             preferred_element_type=jnp.float32)
    z = gamma_ref[0] * yA
    x_ = _soft_threshold(z, theta_ref[0])
    x = x_ + a_par_ref[0] * (x_ / (jnp.abs(x_) + _EPS))

    for i in range(1, _K):
        tvu = theta_ref[i] * vu_ref[i]
        g = x + tvu * x * jnp.exp(-v_ref[i] * jnp.abs(x))
        a = jax.lax.dot_general(g, A, (((1,), (1,)), ((), ())),
                                preferred_element_type=jnp.float32)
        b = a - y
        c = jax.lax.dot_general(b, A, (((1,), (0,)), ((), ())),
                                preferred_element_type=jnp.float32)
        z = x - gamma_ref[i] * c
        x_ = _soft_threshold(z, theta_ref[i])
        dx = x_ - x
        x = x_ + a_par_ref[i] * (dx / (jnp.abs(dx) + _EPS))

    out_ref[...] = x


@jax.jit
def kernel(y, info, A, gamma, theta, a_par, v, vu, theta_init):
    batch, m = y.shape
    n = A.shape[1]
    smem = pl.BlockSpec(memory_space=pltpu.SMEM)
    x = pl.pallas_call(
        _body,
        grid=(batch // _ROWS,),
        in_specs=[
            pl.BlockSpec((_ROWS, m), lambda i: (i, 0)),
            pl.BlockSpec((m, n), lambda i: (0, 0)),
            smem, smem, smem, smem, smem,
        ],
        out_specs=pl.BlockSpec((_ROWS, n), lambda i: (i, 0)),
        out_shape=jax.ShapeDtypeStruct((batch, n), jnp.float32),
        compiler_params=pltpu.CompilerParams(
            dimension_semantics=("parallel",),
            vmem_limit_bytes=100 * 1024 * 1024,
        ),
    )(y, A, gamma, theta, a_par, v, vu)
    zk = jnp.zeros((_K, 1), jnp.float32)
    return x, zk, zk
